# Initial kernel scaffold; baseline (speedup 1.0000x reference)
import functools

import jax
import jax.numpy as jnp
from jax import lax
from jax.experimental import pallas as pl
from jax.experimental.pallas import tpu as pltpu
from jax.experimental.pallas import tpu_sc as plsc

N = 10000
E = 160000
D = 256
BN_EPS = 1e-5

NC = 2
NS = 16
HALF = D // 2
BLK = 128
NBLK = 79
EPT = NBLK * BLK
E_PAD = EPT * NS
N_PAD = N + 16
ZROWS = N_PAD // NS

BR = 1000
R = N // BR


_mesh = plsc.VectorSubcoreMesh(core_axis_name="c", subcore_axis_name="s")


@functools.partial(
    pl.kernel,
    out_type=jax.ShapeDtypeStruct((NC, N_PAD, HALF), jnp.float32),
    mesh=_mesh,
    scratch_types=[
        pltpu.VMEM((NBLK, BLK), jnp.int32),
        pltpu.VMEM((NBLK, BLK), jnp.int32),
        pltpu.VMEM((BLK, HALF), jnp.float32),
        pltpu.VMEM_SHARED((N_PAD, HALF), jnp.float32),
        pltpu.SemaphoreType.DMA,
    ],
)
def _sc_agg(x2_hbm, src_hbm, dst_hbm, zero_hbm, out_hbm,
            src_v, dst_v, rows_v, agg_sh, sem):
    c = lax.axis_index("c")
    s = lax.axis_index("s")
    pltpu.sync_copy(zero_hbm, agg_sh.at[pl.ds(s * ZROWS, ZROWS)])
    pltpu.sync_copy(src_hbm.at[s], src_v)
    pltpu.sync_copy(dst_hbm.at[s], dst_v)
    plsc.subcore_barrier()

    def body(j, carry):
        pltpu.async_copy(x2_hbm.at[c].at[src_v.at[j]], rows_v, sem).wait()
        pltpu.sync_copy(rows_v, agg_sh.at[dst_v.at[j]], add=True)
        return carry

    lax.fori_loop(0, NBLK, body, 0)
    plsc.subcore_barrier()
    pltpu.sync_copy(agg_sh.at[pl.ds(s * ZROWS, ZROWS)],
                    out_hbm.at[c].at[pl.ds(s * ZROWS, ZROWS)])


def _lin_body(x2_ref, agg2_ref, w2_ref, b_ref, h_ref, sums_ref):
    r = pl.program_id(0)
    k = pl.program_id(1)
    xa = x2_ref[0] + agg2_ref[0]
    part = lax.dot_general(xa, w2_ref[0], (((1,), (1,)), ((), ())),
                           preferred_element_type=jnp.float32)

    @pl.when(k == 0)
    def _():
        h_ref[...] = part + b_ref[...]

    @pl.when(k == 1)
    def _():
        h = h_ref[...] + part
        h_ref[...] = h
        s0 = jnp.sum(h, axis=0, keepdims=True)
        s1 = jnp.sum(h * h, axis=0, keepdims=True)
        blk = jnp.concatenate(
            [s0, s1, jnp.zeros((6, D), jnp.float32)], axis=0)

        @pl.when(r == 0)
        def _():
            sums_ref[...] = blk

        @pl.when(r > 0)
        def _():
            sums_ref[...] = sums_ref[...] + blk


def _bn_body(h_ref, sums_ref, g_ref, bt_ref, o_ref):
    mean = sums_ref[0:1, :] * (1.0 / N)
    ex2 = sums_ref[1:2, :] * (1.0 / N)
    var = ex2 - mean * mean
    inv = g_ref[...] * lax.rsqrt(var + BN_EPS)
    o_ref[...] = jnp.maximum((h_ref[...] - mean) * inv + bt_ref[...], 0.0)


@jax.jit
def kernel(x, edge_index, W, b, gamma, beta):
    src = edge_index[0]
    dst = edge_index[1]
    pad = E_PAD - E
    src3 = jnp.concatenate(
        [src, jnp.zeros((pad,), jnp.int32)]).reshape(NS, NBLK, BLK)
    dst3 = jnp.concatenate(
        [dst, jnp.full((pad,), N, jnp.int32)]).reshape(NS, NBLK, BLK)
    x2 = x.reshape(N, NC, HALF).transpose(1, 0, 2)
    zeros_chunk = jnp.zeros((ZROWS, HALF), jnp.float32)

    agg2 = _sc_agg(x2, src3, dst3, zeros_chunk)

    W2 = W.reshape(D, NC, HALF).transpose(1, 0, 2)
    h, sums = pl.pallas_call(
        _lin_body,
        grid=(R, NC),
        in_specs=[
            pl.BlockSpec((1, BR, HALF), lambda r, k: (k, r, 0)),
            pl.BlockSpec((1, BR, HALF), lambda r, k: (k, r, 0)),
            pl.BlockSpec((1, D, HALF), lambda r, k: (k, 0, 0)),
            pl.BlockSpec((1, D), lambda r, k: (0, 0)),
        ],
        out_specs=[
            pl.BlockSpec((BR, D), lambda r, k: (r, 0)),
            pl.BlockSpec((8, D), lambda r, k: (0, 0)),
        ],
        out_shape=[
            jax.ShapeDtypeStruct((N, D), jnp.float32),
            jax.ShapeDtypeStruct((8, D), jnp.float32),
        ],
    )(x2, agg2, W2, b.reshape(1, D))

    out = pl.pallas_call(
        _bn_body,
        grid=(R,),
        in_specs=[
            pl.BlockSpec((BR, D), lambda r: (r, 0)),
            pl.BlockSpec((8, D), lambda r: (0, 0)),
            pl.BlockSpec((1, D), lambda r: (0, 0)),
            pl.BlockSpec((1, D), lambda r: (0, 0)),
        ],
        out_specs=pl.BlockSpec((BR, D), lambda r: (r, 0)),
        out_shape=jax.ShapeDtypeStruct((N, D), jnp.float32),
    )(h, sums, gamma.reshape(1, D), beta.reshape(1, D))
    return out


# speedup vs baseline: 3.8735x; 3.8735x over previous
import functools

import jax
import jax.numpy as jnp
from jax import lax
from jax.experimental import pallas as pl
from jax.experimental.pallas import tpu as pltpu
from jax.experimental.pallas import tpu_sc as plsc

N = 10000
E = 160000
D = 256
BN_EPS = 1e-5

NC = 2
NS = 16
HALF = D // 2
BLK = 128
NBLK = 79
EPT = NBLK * BLK
E_PAD = EPT * NS
ZROWS = 632
N_PAD = ZROWS * NS

BR = 1000
R = N // BR


_mesh = plsc.VectorSubcoreMesh(core_axis_name="c", subcore_axis_name="s")


@functools.partial(
    pl.kernel,
    out_type=jax.ShapeDtypeStruct((NC, N_PAD, HALF), jnp.float32),
    mesh=_mesh,
    scratch_types=[
        pltpu.VMEM((NBLK, BLK), jnp.int32),
        pltpu.VMEM((NBLK, BLK), jnp.int32),
        pltpu.VMEM((BLK, HALF), jnp.float32),
        pltpu.VMEM_SHARED((N_PAD, HALF), jnp.float32),
        pltpu.SemaphoreType.DMA,
    ],
)
def _sc_agg(x2_hbm, src_hbm, dst_hbm, zero_hbm, out_hbm,
            src_v, dst_v, rows_v, agg_sh, sem):
    c = lax.axis_index("c")
    s = lax.axis_index("s")
    base = pl.multiple_of(s * ZROWS, 8)
    pltpu.sync_copy(zero_hbm, agg_sh.at[pl.ds(base, ZROWS)])
    pltpu.sync_copy(src_hbm.at[s], src_v)
    pltpu.sync_copy(dst_hbm.at[s], dst_v)
    plsc.subcore_barrier()

    def body(j, carry):
        pltpu.async_copy(x2_hbm.at[c].at[src_v.at[j]], rows_v, sem).wait()
        pltpu.sync_copy(rows_v, agg_sh.at[dst_v.at[j]], add=True)
        return carry

    lax.fori_loop(0, NBLK, body, 0)
    plsc.subcore_barrier()
    pltpu.sync_copy(agg_sh.at[pl.ds(base, ZROWS)],
                    out_hbm.at[c].at[pl.ds(base, ZROWS)])


def _lin_body(x2_ref, agg2_ref, w2_ref, b_ref, h_ref, sums_ref):
    r = pl.program_id(0)
    k = pl.program_id(1)
    xa = x2_ref[0] + agg2_ref[0]
    part = lax.dot_general(xa, w2_ref[0], (((1,), (1,)), ((), ())),
                           preferred_element_type=jnp.float32)

    @pl.when(k == 0)
    def _():
        h_ref[...] = part + b_ref[...]

    @pl.when(k == 1)
    def _():
        h = h_ref[...] + part
        h_ref[...] = h
        s0 = jnp.sum(h, axis=0, keepdims=True)
        s1 = jnp.sum(h * h, axis=0, keepdims=True)
        blk = jnp.concatenate(
            [s0, s1, jnp.zeros((6, D), jnp.float32)], axis=0)

        @pl.when(r == 0)
        def _():
            sums_ref[...] = blk

        @pl.when(r > 0)
        def _():
            sums_ref[...] = sums_ref[...] + blk


def _bn_body(h_ref, sums_ref, g_ref, bt_ref, o_ref):
    mean = sums_ref[0:1, :] * (1.0 / N)
    ex2 = sums_ref[1:2, :] * (1.0 / N)
    var = ex2 - mean * mean
    inv = g_ref[...] * lax.rsqrt(var + BN_EPS)
    o_ref[...] = jnp.maximum((h_ref[...] - mean) * inv + bt_ref[...], 0.0)


@jax.jit
def kernel(x, edge_index, W, b, gamma, beta):
    src = edge_index[0]
    dst = edge_index[1]
    pad = E_PAD - E
    src3 = jnp.concatenate(
        [src, jnp.zeros((pad,), jnp.int32)]).reshape(NS, NBLK, BLK)
    dst3 = jnp.concatenate(
        [dst, jnp.full((pad,), N, jnp.int32)]).reshape(NS, NBLK, BLK)
    x2 = x.reshape(N, NC, HALF).transpose(1, 0, 2)
    zeros_chunk = jnp.zeros((ZROWS, HALF), jnp.float32)

    agg2 = _sc_agg(x2, src3, dst3, zeros_chunk)

    W2 = W.reshape(D, NC, HALF).transpose(1, 0, 2)
    h, sums = pl.pallas_call(
        _lin_body,
        grid=(R, NC),
        in_specs=[
            pl.BlockSpec((1, BR, HALF), lambda r, k: (k, r, 0)),
            pl.BlockSpec((1, BR, HALF), lambda r, k: (k, r, 0)),
            pl.BlockSpec((1, D, HALF), lambda r, k: (k, 0, 0)),
            pl.BlockSpec((1, D), lambda r, k: (0, 0)),
        ],
        out_specs=[
            pl.BlockSpec((BR, D), lambda r, k: (r, 0)),
            pl.BlockSpec((8, D), lambda r, k: (0, 0)),
        ],
        out_shape=[
            jax.ShapeDtypeStruct((N, D), jnp.float32),
            jax.ShapeDtypeStruct((8, D), jnp.float32),
        ],
    )(x2, agg2, W2, b.reshape(1, D))

    out = pl.pallas_call(
        _bn_body,
        grid=(R,),
        in_specs=[
            pl.BlockSpec((BR, D), lambda r: (r, 0)),
            pl.BlockSpec((8, D), lambda r: (0, 0)),
            pl.BlockSpec((1, D), lambda r: (0, 0)),
            pl.BlockSpec((1, D), lambda r: (0, 0)),
        ],
        out_specs=pl.BlockSpec((BR, D), lambda r: (r, 0)),
        out_shape=jax.ShapeDtypeStruct((N, D), jnp.float32),
    )(h, sums, gamma.reshape(1, D), beta.reshape(1, D))
    return out
